# baseline (speedup 1.0000x reference)
import functools

import jax
import jax.numpy as jnp
from jax import lax
from jax.experimental import pallas as pl
from jax.experimental.pallas import tpu as pltpu
from jax.experimental.pallas import tpu_sc as plsc

N = 10000
E = 320000
D = 64
F = 2 * D


_ROWS_BLK = 1000


def _norm_body(x_d, h_d, x_g, h_g, d_out, g_out):
    fd = jnp.concatenate([x_d[...], h_d[...]], axis=1)
    fg = jnp.concatenate([x_g[...], h_g[...]], axis=1)
    d_out[...] = fd / jnp.sqrt(jnp.sum(fd * fd, axis=1, keepdims=True))
    g_out[...] = fg / jnp.sqrt(jnp.sum(fg * fg, axis=1, keepdims=True))


def _normalized_tables(x_d, h_d, x_g, h_g):
    bs_in = pl.BlockSpec((_ROWS_BLK, D), lambda i: (i, 0))
    bs_out = pl.BlockSpec((_ROWS_BLK, F), lambda i: (i, 0))
    return pl.pallas_call(
        _norm_body,
        grid=(N // _ROWS_BLK,),
        in_specs=[bs_in] * 4,
        out_specs=[bs_out, bs_out],
        out_shape=[jax.ShapeDtypeStruct((N, F), jnp.float32)] * 2,
    )(x_d, h_d, x_g, h_g)


_info = plsc.get_sparse_core_info()
_NC, _NS, _L = _info.num_cores, _info.num_subcores, _info.num_lanes
_NW = _NC * _NS
_EPW = E // _NW
_C = 80
_NCHUNK = _EPW // _C
_GROUPS = _C // _L
_NBUF = 4

_BITREV = [int("{:04b}".format(i)[::-1], 2) for i in range(16)]

_mesh = plsc.VectorSubcoreMesh(core_axis_name="c", subcore_axis_name="s")


@functools.partial(
    pl.kernel,
    mesh=_mesh,
    out_type=jax.ShapeDtypeStruct((E,), jnp.float32),
    scratch_types=(
        [pltpu.VMEM((_EPW,), jnp.int32)] * 2
        + [pltpu.VMEM((_C, F), jnp.float32)] * (2 * _NBUF)
        + [pltpu.VMEM((_EPW,), jnp.float32)]
        + [pltpu.VMEM((_C, _L), jnp.float32)]
        + [pltpu.SemaphoreType.DMA] * (2 * _NBUF)
    ),
)
def _edge_scores(g_hbm, d_hbm, src_hbm, dst_hbm, out_hbm, *refs):
    src_v, dst_v = refs[0], refs[1]
    gbuf = refs[2:2 + _NBUF]
    dbuf = refs[2 + _NBUF:2 + 2 * _NBUF]
    out_v = refs[2 + 2 * _NBUF]
    acc_v = refs[3 + 2 * _NBUF]
    sg = refs[4 + 2 * _NBUF:4 + 3 * _NBUF]
    sd = refs[4 + 3 * _NBUF:4 + 4 * _NBUF]

    wid = lax.axis_index("s") * _NC + lax.axis_index("c")
    base = wid * _EPW
    pltpu.sync_copy(src_hbm.at[pl.ds(base, _EPW)], src_v)
    pltpu.sync_copy(dst_hbm.at[pl.ds(base, _EPW)], dst_v)

    def start(c, b):
        idx_s = src_v.at[pl.ds(c * _C, _C)]
        idx_d = dst_v.at[pl.ds(c * _C, _C)]
        pltpu.async_copy(g_hbm.at[idx_s], gbuf[b], sg[b])
        pltpu.async_copy(d_hbm.at[idx_d], dbuf[b], sd[b])

    def wait(c, b):
        idx_s = src_v.at[pl.ds(c * _C, _C)]
        idx_d = dst_v.at[pl.ds(c * _C, _C)]
        pltpu.make_async_copy(g_hbm.at[idx_s], gbuf[b], sg[b]).wait()
        pltpu.make_async_copy(d_hbm.at[idx_d], dbuf[b], sd[b]).wait()

    def compute(c, b):
        gr = gbuf[b]
        dr = dbuf[b]

        lanes = lax.iota(jnp.int32, _L)

        dnums = lax.GatherDimensionNumbers(
            offset_dims=(), collapsed_slice_dims=(0,), start_index_map=(0,))

        def vshuf(x, idx):
            return lax.gather(
                x, idx[:, None], dnums, (1,),
                mode=lax.GatherScatterMode.PROMISE_IN_BOUNDS)

        if True:
            return
        @plsc.parallel_loop(0, _C, unroll=4)
        def _edge(e):
            prods = []
            for j in range(F // _L):
                gv = gr[e, pl.ds(j * _L, _L)]
                dv = dr[e, pl.ds(j * _L, _L)]
                prods.append(gv * dv)
            while len(prods) > 1:
                prods = [a + b2 for a, b2 in zip(prods[::2], prods[1::2])]
            acc_v[e, :] = prods[0]

        @plsc.parallel_loop(0, _GROUPS)
        def _group(g):
            vs = [acc_v[g * _L + _BITREV[i], :] for i in range(_L)]
            for d in (8, 4, 2, 1):
                m = (lanes & d) == 0
                sh = lanes ^ d
                vs = [jnp.where(m, a, vshuf(b2, sh))
                      + jnp.where(m, vshuf(a, sh), b2)
                      for a, b2 in zip(vs[::2], vs[1::2])]
            out_v[pl.ds(c * _C + g * _L, _L)] = vs[0]

    for b0 in range(_NBUF - 1):
        start(b0, b0)

    def chunk_ring(i, carry):
        c0 = i * _NBUF
        for b in range(_NBUF):
            c = c0 + b

            @pl.when(c < _NCHUNK)
            def _process():
                @pl.when(c + _NBUF - 1 < _NCHUNK)
                def _prefetch():
                    start(c + _NBUF - 1, (b + _NBUF - 1) % _NBUF)

                wait(c, b)
                compute(c, b)

        return carry

    lax.fori_loop(0, (_NCHUNK + _NBUF - 1) // _NBUF, chunk_ring, 0)
    pltpu.sync_copy(out_v, out_hbm.at[pl.ds(base, _EPW)])


def kernel(h_disease, h_gene, x_disease, x_gene, edge_index):
    d_tab, g_tab = _normalized_tables(x_disease, h_disease, x_gene, h_gene)
    src = edge_index[0].astype(jnp.int32)
    dst = edge_index[1].astype(jnp.int32)
    scores = _edge_scores(g_tab, d_tab, src, dst)
    return scores.reshape(E, 1)


# speedup vs baseline: 1.0146x; 1.0146x over previous
import functools

import jax
import jax.numpy as jnp
from jax import lax
from jax.experimental import pallas as pl
from jax.experimental.pallas import tpu as pltpu
from jax.experimental.pallas import tpu_sc as plsc

N = 10000
E = 320000
D = 64
F = 2 * D


_ROWS_BLK = 1000


def _norm_body(x_d, h_d, x_g, h_g, d_out, g_out):
    fd = jnp.concatenate([x_d[...], h_d[...]], axis=1)
    fg = jnp.concatenate([x_g[...], h_g[...]], axis=1)
    d_out[...] = fd / jnp.sqrt(jnp.sum(fd * fd, axis=1, keepdims=True))
    g_out[...] = fg / jnp.sqrt(jnp.sum(fg * fg, axis=1, keepdims=True))


def _normalized_tables(x_d, h_d, x_g, h_g):
    bs_in = pl.BlockSpec((_ROWS_BLK, D), lambda i: (i, 0))
    bs_out = pl.BlockSpec((_ROWS_BLK, F), lambda i: (i, 0))
    return pl.pallas_call(
        _norm_body,
        grid=(N // _ROWS_BLK,),
        in_specs=[bs_in] * 4,
        out_specs=[bs_out, bs_out],
        out_shape=[jax.ShapeDtypeStruct((N, F), jnp.float32)] * 2,
    )(x_d, h_d, x_g, h_g)


_info = plsc.get_sparse_core_info()
_NC, _NS, _L = _info.num_cores, _info.num_subcores, _info.num_lanes
_NW = _NC * _NS
_EPW = E // _NW
_C = 80
_NCHUNK = _EPW // _C
_GROUPS = _C // _L
_NBUF = 4

_BITREV = [int("{:04b}".format(i)[::-1], 2) for i in range(16)]

_mesh = plsc.VectorSubcoreMesh(core_axis_name="c", subcore_axis_name="s")


@functools.partial(
    pl.kernel,
    mesh=_mesh,
    out_type=jax.ShapeDtypeStruct((E,), jnp.float32),
    scratch_types=(
        [pltpu.VMEM((_EPW,), jnp.int32)] * 2
        + [pltpu.VMEM((_C, F), jnp.float32)] * (2 * _NBUF)
        + [pltpu.VMEM((_EPW,), jnp.float32)]
        + [pltpu.VMEM((_C, _L), jnp.float32)]
        + [pltpu.SemaphoreType.DMA] * (2 * _NBUF)
    ),
)
def _edge_scores(g_hbm, d_hbm, src_hbm, dst_hbm, out_hbm, *refs):
    src_v, dst_v = refs[0], refs[1]
    gbuf = refs[2:2 + _NBUF]
    dbuf = refs[2 + _NBUF:2 + 2 * _NBUF]
    out_v = refs[2 + 2 * _NBUF]
    acc_v = refs[3 + 2 * _NBUF]
    sg = refs[4 + 2 * _NBUF:4 + 3 * _NBUF]
    sd = refs[4 + 3 * _NBUF:4 + 4 * _NBUF]

    wid = lax.axis_index("s") * _NC + lax.axis_index("c")
    base = wid * _EPW
    pltpu.sync_copy(src_hbm.at[pl.ds(base, _EPW)], src_v)
    pltpu.sync_copy(dst_hbm.at[pl.ds(base, _EPW)], dst_v)

    def start(c, b):
        idx_s = src_v.at[pl.ds(c * _C, _C)]
        idx_d = dst_v.at[pl.ds(c * _C, _C)]
        pltpu.async_copy(g_hbm.at[idx_s], gbuf[b], sg[b])
        pltpu.async_copy(d_hbm.at[idx_d], dbuf[b], sd[b])

    def wait(c, b):
        idx_s = src_v.at[pl.ds(c * _C, _C)]
        idx_d = dst_v.at[pl.ds(c * _C, _C)]
        pltpu.make_async_copy(g_hbm.at[idx_s], gbuf[b], sg[b]).wait()
        pltpu.make_async_copy(d_hbm.at[idx_d], dbuf[b], sd[b]).wait()

    def compute(c, b):
        gr = gbuf[b]
        dr = dbuf[b]

        lanes = lax.iota(jnp.int32, _L)

        dnums = lax.GatherDimensionNumbers(
            offset_dims=(), collapsed_slice_dims=(0,), start_index_map=(0,))

        def vshuf(x, idx):
            return lax.gather(
                x, idx[:, None], dnums, (1,),
                mode=lax.GatherScatterMode.PROMISE_IN_BOUNDS)

        @plsc.parallel_loop(0, _C, unroll=4)
        def _edge(e):
            prods = []
            for j in range(F // _L):
                gv = gr[e, pl.ds(j * _L, _L)]
                dv = dr[e, pl.ds(j * _L, _L)]
                prods.append(gv * dv)
            while len(prods) > 1:
                prods = [a + b2 for a, b2 in zip(prods[::2], prods[1::2])]
            acc_v[e, :] = prods[0]

        @plsc.parallel_loop(0, _GROUPS)
        def _group(g):
            vs = [acc_v[g * _L + _BITREV[i], :] for i in range(_L)]
            for d in (8, 4, 2, 1):
                m = (lanes & d) == 0
                sh = lanes ^ d
                vs = [jnp.where(m, a, vshuf(b2, sh))
                      + jnp.where(m, vshuf(a, sh), b2)
                      for a, b2 in zip(vs[::2], vs[1::2])]
            out_v[pl.ds(c * _C + g * _L, _L)] = vs[0]

    for b0 in range(_NBUF - 1):
        start(b0, b0)

    def chunk_ring(i, carry):
        c0 = i * _NBUF
        for b in range(_NBUF):
            c = c0 + b

            @pl.when(c < _NCHUNK)
            def _process():
                @pl.when(c + _NBUF - 1 < _NCHUNK)
                def _prefetch():
                    start(c + _NBUF - 1, (b + _NBUF - 1) % _NBUF)

                wait(c, b)
                compute(c, b)

        return carry

    lax.fori_loop(0, (_NCHUNK + _NBUF - 1) // _NBUF, chunk_ring, 0)
    pltpu.sync_copy(out_v, out_hbm.at[pl.ds(base, _EPW)])


def kernel(h_disease, h_gene, x_disease, x_gene, edge_index):
    d_tab, g_tab = _normalized_tables(x_disease, h_disease, x_gene, h_gene)
    src = edge_index[0].astype(jnp.int32)
    dst = edge_index[1].astype(jnp.int32)
    scores = _edge_scores(g_tab, d_tab, src, dst)
    return scores.reshape(E, 1)
